# baseline (speedup 1.0000x reference)
import jax
import jax.numpy as jnp
from jax.experimental import pallas as pl


def _add_kernel(patch_ref, pos_ref, out_ref):
    out_ref[...] = patch_ref[...] + pos_ref[...]


def kernel(patch, pos_table):
    B, N, D = patch.shape
    return pl.pallas_call(
        _add_kernel,
        grid=(B,),
        in_specs=[
            pl.BlockSpec((1, N, D), lambda b: (b, 0, 0)),
            pl.BlockSpec((N, D), lambda b: (0, 0)),
        ],
        out_specs=pl.BlockSpec((1, N, D), lambda b: (b, 0, 0)),
        out_shape=jax.ShapeDtypeStruct((B, N, D), patch.dtype),
    )(patch, pos_table)


# speedup vs baseline: 1.1880x; 1.1880x over previous
import jax
import jax.numpy as jnp
from jax.experimental import pallas as pl


def _add_kernel(patch_ref, pos_ref, out_ref):
    out_ref[...] = patch_ref[...] + pos_ref[...]


_BB = 4


def kernel(patch, pos_table):
    B, N, D = patch.shape
    return pl.pallas_call(
        _add_kernel,
        grid=(B // _BB,),
        in_specs=[
            pl.BlockSpec((_BB, N, D), lambda b: (b, 0, 0)),
            pl.BlockSpec((N, D), lambda b: (0, 0)),
        ],
        out_specs=pl.BlockSpec((_BB, N, D), lambda b: (b, 0, 0)),
        out_shape=jax.ShapeDtypeStruct((B, N, D), patch.dtype),
    )(patch, pos_table)


# speedup vs baseline: 1.2061x; 1.0152x over previous
import jax
import jax.numpy as jnp
from jax.experimental import pallas as pl


def _add_kernel(patch_ref, pos_ref, out_ref):
    out_ref[...] = patch_ref[...] + pos_ref[...]


_BB = 8


def kernel(patch, pos_table):
    B, N, D = patch.shape
    return pl.pallas_call(
        _add_kernel,
        grid=(B // _BB,),
        in_specs=[
            pl.BlockSpec((_BB, N, D), lambda b: (b, 0, 0)),
            pl.BlockSpec((N, D), lambda b: (0, 0)),
        ],
        out_specs=pl.BlockSpec((_BB, N, D), lambda b: (b, 0, 0)),
        out_shape=jax.ShapeDtypeStruct((B, N, D), patch.dtype),
    )(patch, pos_table)
